# Initial kernel scaffold; baseline (speedup 1.0000x reference)
import jax
import jax.numpy as jnp
from jax.experimental import pallas as pl


def kernel(q, k, v):
    raise NotImplementedError("write your pallas kernel here")


# speedup vs baseline: 29.2898x; 29.2898x over previous
import functools

import jax
import jax.numpy as jnp
from jax.experimental import pallas as pl
from jax.experimental.pallas import tpu as pltpu

_TOPK = 64


def _attn_kernel(q_ref, k_ref, v_ref, o_ref, *, topk):
    q = q_ref[0, 0]
    k = k_ref[0, 0]
    v = v_ref[0, 0]
    d = q.shape[-1]
    scale = 1.0 / (float(d) ** 0.5)

    s = jax.lax.dot_general(
        q.astype(jnp.bfloat16), k.astype(jnp.bfloat16), (((1,), (1,)), ((), ())),
        preferred_element_type=jnp.float32,
    ) * scale

    i = jax.lax.bitcast_convert_type(s, jnp.int32)
    ikey = i ^ ((i >> 31) & jnp.int32(0x7FFFFFFF))

    bq = s.shape[0]
    lo0 = jnp.full((bq, 1), jnp.int32(-2147483648))
    hi0 = jnp.full((bq, 1), jnp.int32(2147483647))

    def body(_, carry):
        lo, hi = carry
        x = lo ^ hi
        mid = (lo & hi) + (x >> 1) + (x & 1)
        cnt = jnp.sum((ikey >= mid).astype(jnp.int32), axis=-1, keepdims=True)
        pred = cnt >= topk
        return jnp.where(pred, mid, lo), jnp.where(pred, hi, mid - 1)

    lo, _ = jax.lax.fori_loop(0, 32, body, (lo0, hi0))
    mask = ikey >= lo

    m = jnp.max(s, axis=-1, keepdims=True)
    p = jnp.where(mask, jnp.exp(s - m), 0.0)
    z = jnp.sum(p, axis=-1, keepdims=True)
    w = (p / z).astype(jnp.bfloat16)
    o = jax.lax.dot_general(
        w, v.astype(jnp.bfloat16), (((1,), (0,)), ((), ())),
        preferred_element_type=jnp.float32,
    )
    o_ref[0, 0] = o


def kernel(q, k, v):
    b, h, s_len, d = q.shape
    bq = min(512, s_len)
    n_qb = s_len // bq
    grid = (b * h, n_qb)

    out = pl.pallas_call(
        functools.partial(_attn_kernel, topk=_TOPK),
        grid=grid,
        in_specs=[
            pl.BlockSpec((1, 1, bq, d), lambda hh, qb: (0, hh, qb, 0)),
            pl.BlockSpec((1, 1, s_len, d), lambda hh, qb: (0, hh, 0, 0)),
            pl.BlockSpec((1, 1, s_len, d), lambda hh, qb: (0, hh, 0, 0)),
        ],
        out_specs=pl.BlockSpec((1, 1, bq, d), lambda hh, qb: (0, hh, qb, 0)),
        out_shape=jax.ShapeDtypeStruct((b, h, s_len, d), q.dtype),
        compiler_params=pltpu.CompilerParams(
            dimension_semantics=("arbitrary", "arbitrary"),
        ),
    )(q, k, v)
    return out


# speedup vs baseline: 45.0521x; 1.5382x over previous
import functools

import jax
import jax.numpy as jnp
from jax.experimental import pallas as pl
from jax.experimental.pallas import tpu as pltpu

_TOPK = 64


def _attn_kernel(q_ref, k_ref, v_ref, o_ref, *, topk):
    q = q_ref[0, 0]
    k = k_ref[0, 0]
    v = v_ref[0, 0]
    d = q.shape[-1]
    scale = 1.0 / (float(d) ** 0.5)

    s = jax.lax.dot_general(
        k.astype(jnp.bfloat16), q.astype(jnp.bfloat16), (((1,), (1,)), ((), ())),
        preferred_element_type=jnp.float32,
    ) * scale

    i = jax.lax.bitcast_convert_type(s, jnp.int32)
    ikey = i ^ ((i >> 31) & jnp.int32(0x7FFFFFFF))

    bq = s.shape[1]
    lo0 = jnp.full((1, bq), jnp.int32(-2147483648))
    hi0 = jnp.full((1, bq), jnp.int32(2147483647))

    def body(_, carry):
        lo, hi = carry
        x = lo ^ hi
        mid = (lo & hi) + (x >> 1) + (x & 1)
        cnt = jnp.sum((ikey >= mid).astype(jnp.int32), axis=0, keepdims=True)
        pred = cnt >= topk
        return jnp.where(pred, mid, lo), jnp.where(pred, hi, mid - 1)

    lo, _ = jax.lax.fori_loop(0, 32, body, (lo0, hi0))
    mask = ikey >= lo

    t_f = jax.lax.bitcast_convert_type(lo ^ ((lo >> 31) & jnp.int32(0x7FFFFFFF)),
                                       jnp.float32)
    p = jnp.where(mask, jnp.exp(s - t_f), 0.0)
    z = jnp.sum(p, axis=0, keepdims=True)
    o = jax.lax.dot_general(
        p.astype(jnp.bfloat16), v.astype(jnp.bfloat16), (((0,), (0,)), ((), ())),
        preferred_element_type=jnp.float32,
    )
    zt = jnp.swapaxes(z, 0, 1)
    o_ref[0, 0] = o / zt


def kernel(q, k, v):
    b, h, s_len, d = q.shape
    bq = min(1024, s_len)
    n_qb = s_len // bq
    grid = (b * h, n_qb)

    out = pl.pallas_call(
        functools.partial(_attn_kernel, topk=_TOPK),
        grid=grid,
        in_specs=[
            pl.BlockSpec((1, 1, bq, d), lambda hh, qb: (0, hh, qb, 0)),
            pl.BlockSpec((1, 1, s_len, d), lambda hh, qb: (0, hh, 0, 0)),
            pl.BlockSpec((1, 1, s_len, d), lambda hh, qb: (0, hh, 0, 0)),
        ],
        out_specs=pl.BlockSpec((1, 1, bq, d), lambda hh, qb: (0, hh, qb, 0)),
        out_shape=jax.ShapeDtypeStruct((b, h, s_len, d), q.dtype),
        compiler_params=pltpu.CompilerParams(
            dimension_semantics=("arbitrary", "arbitrary"),
        ),
    )(q, k, v)
    return out
